# baseline (speedup 1.0000x reference)
import functools

import jax
import jax.numpy as jnp
from jax.experimental import pallas as pl
from jax.experimental.pallas import tpu as pltpu

_MINE, _MAXE = 0.1, 1.9
_TRUNC = 0.1
_EPS = 1e-6


def _prep_kernel(scale_ref, exps_ref, rot_ref, taper_ref, trans_ref,
                 out_ref):
    scale = jnp.exp(scale_ref[...]) + 1e-6
    inv_s = 1.0 / scale
    isx = inv_s[:, 0:1]
    isy = inv_s[:, 1:2]
    isz = inv_s[:, 2:3]

    e = jax.nn.sigmoid(exps_ref[...]) * (_MAXE - _MINE) + _MINE
    e1 = e[:, 0:1]
    e2 = e[:, 1:2]
    p2 = 2.0 / e2
    p21 = e2 / e1
    p1 = 2.0 / e1
    ph = -0.5 * e1

    q = rot_ref[...]
    q = q / (jnp.sqrt(jnp.sum(q * q, axis=-1, keepdims=True)) + 1e-12)
    qw = q[:, 0:1]
    qx = q[:, 1:2]
    qy = q[:, 2:3]
    qz = q[:, 3:4]
    r00 = 1 - 2 * (qy * qy + qz * qz)
    r01 = 2 * (qx * qy - qw * qz)
    r02 = 2 * (qx * qz + qw * qy)
    r10 = 2 * (qx * qy + qw * qz)
    r11 = 1 - 2 * (qx * qx + qz * qz)
    r12 = 2 * (qy * qz - qw * qx)
    r20 = 2 * (qx * qz - qw * qy)
    r21 = 2 * (qy * qz + qw * qx)
    r22 = 1 - 2 * (qx * qx + qy * qy)

    taper = jnp.tanh(taper_ref[...])
    cx = taper[:, 0:1] * isz
    cy = taper[:, 1:2] * isz

    tx = trans_ref[:, 0:1]
    ty = trans_ref[:, 1:2]
    tz = trans_ref[:, 2:3]

    out_ref[...] = jnp.concatenate([
        r00, r10, r20, r01, r11, r21, r02, r12, r22,
        tx, ty, tz, isx * isx, isy * isy, isz * isz,
        1.0 / e2, p21, 1.0 / e1, ph, cx, cy,
    ], axis=1)


_CH = 64


def _sdf_block_kernel(points_ref, pk_ref, out_ref):
    pk = pk_ref[...]
    px = points_ref[0:1, :]
    py = points_ref[1:2, :]
    pz = points_ref[2:3, :]

    n = pk.shape[0]
    for k in range(0, n, _CH):
        col = lambda j: pk[k:k + _CH, j:j + 1]
        r00, r10, r20 = col(0), col(1), col(2)
        r01, r11, r21 = col(3), col(4), col(5)
        r02, r12, r22 = col(6), col(7), col(8)
        tx, ty, tz = col(9), col(10), col(11)
        isx2, isy2, isz2 = col(12), col(13), col(14)
        q2, p21, q1, ph = col(15), col(16), col(17), col(18)
        cx, cy = col(19), col(20)

        d0 = px - tx
        d1 = py - ty
        d2 = pz - tz
        x0 = r00 * d0 + r10 * d1 + r20 * d2
        x1 = r01 * d0 + r11 * d1 + r21 * d2
        x2 = r02 * d0 + r12 * d1 + r22 * d2

        sq0 = x0 * x0
        sq1 = x1 * x1
        sq2 = x2 * x2
        hs = 0.5 * jnp.log2(sq0 + sq1 + sq2)

        gx = cx * x2 + 1.0
        gy = cy * x2 + 1.0
        af2x = jnp.maximum(gx * gx, _EPS * _EPS)
        af2y = jnp.maximum(gy * gy, _EPS * _EPS)

        lx = jnp.log2(sq0 * isx2) - jnp.log2(af2x)
        ly = jnp.log2(sq1 * isy2) - jnp.log2(af2y)
        lz = jnp.log2(sq2 * isz2)

        A = jnp.exp2(q2 * lx) + jnp.exp2(q2 * ly)
        B = jnp.exp2(p21 * jnp.log2(A)) + jnp.exp2(q1 * lz)
        sdf = (jnp.exp2(hs) + _EPS) - jnp.exp2(hs + ph * jnp.log2(B))
        out_ref[k:k + _CH, :] = jnp.clip(sdf, -_TRUNC, _TRUNC)


@functools.partial(jax.jit, static_argnames=())
def kernel(points, raw_scale, raw_exponents, raw_rotation, raw_tapering,
           translation):
    N = raw_scale.shape[0]
    M = points.shape[1]
    MB = 1024
    grid = (pl.cdiv(M, MB),)

    pk = pl.pallas_call(
        _prep_kernel,
        out_shape=jax.ShapeDtypeStruct((N, 21), jnp.float32),
    )(raw_scale, raw_exponents, raw_rotation, raw_tapering, translation)

    full = lambda shape: pl.BlockSpec(shape, lambda i: (0, 0))
    out = pl.pallas_call(
        _sdf_block_kernel,
        grid=grid,
        in_specs=[
            pl.BlockSpec((3, MB), lambda i: (0, i)),
            full((N, 21)),
        ],
        out_specs=pl.BlockSpec((N, MB), lambda i: (0, i)),
        out_shape=jax.ShapeDtypeStruct((N, M), jnp.float32),
        compiler_params=pltpu.CompilerParams(
            dimension_semantics=("parallel",),
        ),
    )(points, pk)
    return out


# speedup vs baseline: 1.0063x; 1.0063x over previous
import functools

import jax
import jax.numpy as jnp
from jax.experimental import pallas as pl
from jax.experimental.pallas import tpu as pltpu

_MINE, _MAXE = 0.1, 1.9
_TRUNC = 0.1
_EPS = 1e-6


def _prep_kernel(scale_ref, exps_ref, rot_ref, taper_ref, trans_ref,
                 out_ref):
    scale = jnp.exp(scale_ref[...]) + 1e-6
    inv_s = 1.0 / scale
    isx = inv_s[:, 0:1]
    isy = inv_s[:, 1:2]
    isz = inv_s[:, 2:3]

    e = jax.nn.sigmoid(exps_ref[...]) * (_MAXE - _MINE) + _MINE
    e1 = e[:, 0:1]
    e2 = e[:, 1:2]
    p2 = 2.0 / e2
    p21 = e2 / e1
    p1 = 2.0 / e1
    ph = -0.5 * e1

    q = rot_ref[...]
    q = q / (jnp.sqrt(jnp.sum(q * q, axis=-1, keepdims=True)) + 1e-12)
    qw = q[:, 0:1]
    qx = q[:, 1:2]
    qy = q[:, 2:3]
    qz = q[:, 3:4]
    r00 = 1 - 2 * (qy * qy + qz * qz)
    r01 = 2 * (qx * qy - qw * qz)
    r02 = 2 * (qx * qz + qw * qy)
    r10 = 2 * (qx * qy + qw * qz)
    r11 = 1 - 2 * (qx * qx + qz * qz)
    r12 = 2 * (qy * qz - qw * qx)
    r20 = 2 * (qx * qz - qw * qy)
    r21 = 2 * (qy * qz + qw * qx)
    r22 = 1 - 2 * (qx * qx + qy * qy)

    taper = jnp.tanh(taper_ref[...])
    cx = taper[:, 0:1] * isz
    cy = taper[:, 1:2] * isz

    tx = trans_ref[:, 0:1]
    ty = trans_ref[:, 1:2]
    tz = trans_ref[:, 2:3]

    out_ref[...] = jnp.concatenate([
        r00, r10, r20, r01, r11, r21, r02, r12, r22,
        tx, ty, tz, isx * isx, isy * isy, isz * isz,
        1.0 / e2, p21, 1.0 / e1, ph, cx, cy,
    ], axis=1)


_CH = 32


def _sdf_block_kernel(points_ref, pk_ref, out_ref):
    pk = pk_ref[...]
    px = points_ref[0:1, :]
    py = points_ref[1:2, :]
    pz = points_ref[2:3, :]

    n = pk.shape[0]
    for k in range(0, n, _CH):
        col = lambda j: pk[k:k + _CH, j:j + 1]
        r00, r10, r20 = col(0), col(1), col(2)
        r01, r11, r21 = col(3), col(4), col(5)
        r02, r12, r22 = col(6), col(7), col(8)
        tx, ty, tz = col(9), col(10), col(11)
        isx2, isy2, isz2 = col(12), col(13), col(14)
        q2, p21, q1, ph = col(15), col(16), col(17), col(18)
        cx, cy = col(19), col(20)

        d0 = px - tx
        d1 = py - ty
        d2 = pz - tz
        x0 = r00 * d0 + r10 * d1 + r20 * d2
        x1 = r01 * d0 + r11 * d1 + r21 * d2
        x2 = r02 * d0 + r12 * d1 + r22 * d2

        sq0 = x0 * x0
        sq1 = x1 * x1
        sq2 = x2 * x2
        hs = 0.5 * jnp.log2(sq0 + sq1 + sq2)

        gx = cx * x2 + 1.0
        gy = cy * x2 + 1.0
        af2x = jnp.maximum(gx * gx, _EPS * _EPS)
        af2y = jnp.maximum(gy * gy, _EPS * _EPS)

        lx = jnp.log2(sq0 * isx2) - jnp.log2(af2x)
        ly = jnp.log2(sq1 * isy2) - jnp.log2(af2y)
        lz = jnp.log2(sq2 * isz2)

        A = jnp.exp2(q2 * lx) + jnp.exp2(q2 * ly)
        B = jnp.exp2(p21 * jnp.log2(A)) + jnp.exp2(q1 * lz)
        sdf = (jnp.exp2(hs) + _EPS) - jnp.exp2(hs + ph * jnp.log2(B))
        out_ref[k:k + _CH, :] = jnp.clip(sdf, -_TRUNC, _TRUNC)


@functools.partial(jax.jit, static_argnames=())
def kernel(points, raw_scale, raw_exponents, raw_rotation, raw_tapering,
           translation):
    N = raw_scale.shape[0]
    M = points.shape[1]
    MB = 2048
    grid = (pl.cdiv(M, MB),)

    pk = pl.pallas_call(
        _prep_kernel,
        out_shape=jax.ShapeDtypeStruct((N, 21), jnp.float32),
    )(raw_scale, raw_exponents, raw_rotation, raw_tapering, translation)

    full = lambda shape: pl.BlockSpec(shape, lambda i: (0, 0))
    out = pl.pallas_call(
        _sdf_block_kernel,
        grid=grid,
        in_specs=[
            pl.BlockSpec((3, MB), lambda i: (0, i)),
            full((N, 21)),
        ],
        out_specs=pl.BlockSpec((N, MB), lambda i: (0, i)),
        out_shape=jax.ShapeDtypeStruct((N, M), jnp.float32),
        compiler_params=pltpu.CompilerParams(
            dimension_semantics=("parallel",),
        ),
    )(points, pk)
    return out


# speedup vs baseline: 1.0576x; 1.0510x over previous
import functools

import jax
import jax.numpy as jnp
from jax.experimental import pallas as pl
from jax.experimental.pallas import tpu as pltpu

_MINE, _MAXE = 0.1, 1.9
_TRUNC = 0.1
_EPS = 1e-6


def _prep_kernel(scale_ref, exps_ref, rot_ref, taper_ref, trans_ref,
                 out_ref):
    scale = jnp.exp(scale_ref[...]) + 1e-6
    inv_s = 1.0 / scale
    isx = inv_s[:, 0:1]
    isy = inv_s[:, 1:2]
    isz = inv_s[:, 2:3]

    e = jax.nn.sigmoid(exps_ref[...]) * (_MAXE - _MINE) + _MINE
    e1 = e[:, 0:1]
    e2 = e[:, 1:2]
    p2 = 2.0 / e2
    p21 = e2 / e1
    p1 = 2.0 / e1
    ph = -0.5 * e1

    q = rot_ref[...]
    q = q / (jnp.sqrt(jnp.sum(q * q, axis=-1, keepdims=True)) + 1e-12)
    qw = q[:, 0:1]
    qx = q[:, 1:2]
    qy = q[:, 2:3]
    qz = q[:, 3:4]
    r00 = 1 - 2 * (qy * qy + qz * qz)
    r01 = 2 * (qx * qy - qw * qz)
    r02 = 2 * (qx * qz + qw * qy)
    r10 = 2 * (qx * qy + qw * qz)
    r11 = 1 - 2 * (qx * qx + qz * qz)
    r12 = 2 * (qy * qz - qw * qx)
    r20 = 2 * (qx * qz - qw * qy)
    r21 = 2 * (qy * qz + qw * qx)
    r22 = 1 - 2 * (qx * qx + qy * qy)

    taper = jnp.tanh(taper_ref[...])
    cx = taper[:, 0:1] * isz
    cy = taper[:, 1:2] * isz

    tx = trans_ref[:, 0:1]
    ty = trans_ref[:, 1:2]
    tz = trans_ref[:, 2:3]

    out_ref[...] = jnp.concatenate([
        r00, r10, r20, r01, r11, r21, r02, r12, r22,
        tx, ty, tz, isx * isx, isy * isy, isz * isz,
        1.0 / e2, p21, 1.0 / e1, ph, cx, cy,
    ], axis=1)


_CH = 32


def _sdf_block_kernel(points_ref, pk_ref, out_ref):
    pk = pk_ref[...]
    px = points_ref[0:1, :]
    py = points_ref[1:2, :]
    pz = points_ref[2:3, :]

    n = pk.shape[0]
    for k in range(0, n, _CH):
        col = lambda j: pk[k:k + _CH, j:j + 1]
        r00, r10, r20 = col(0), col(1), col(2)
        r01, r11, r21 = col(3), col(4), col(5)
        r02, r12, r22 = col(6), col(7), col(8)
        tx, ty, tz = col(9), col(10), col(11)
        isx2, isy2, isz2 = col(12), col(13), col(14)
        q2, p21, q1, ph = col(15), col(16), col(17), col(18)
        cx, cy = col(19), col(20)

        d0 = px - tx
        d1 = py - ty
        d2 = pz - tz
        x0 = r00 * d0 + r10 * d1 + r20 * d2
        x1 = r01 * d0 + r11 * d1 + r21 * d2
        x2 = r02 * d0 + r12 * d1 + r22 * d2

        sq0 = x0 * x0
        sq1 = x1 * x1
        sq2 = x2 * x2
        hs = 0.5 * jnp.log2(sq0 + sq1 + sq2)

        gx = cx * x2 + 1.0
        gy = cy * x2 + 1.0

        lx = jnp.log2(sq0 * isx2) - jnp.log2(gx * gx)
        ly = jnp.log2(sq1 * isy2) - jnp.log2(gy * gy)
        lz = jnp.log2(sq2 * isz2)

        A = jnp.exp2(q2 * lx) + jnp.exp2(q2 * ly)
        B = jnp.exp2(p21 * jnp.log2(A)) + jnp.exp2(q1 * lz)
        sdf = jnp.exp2(hs) * (1.0 - jnp.exp2(ph * jnp.log2(B)))
        out_ref[k:k + _CH, :] = jnp.clip(sdf, -_TRUNC, _TRUNC)


@functools.partial(jax.jit, static_argnames=())
def kernel(points, raw_scale, raw_exponents, raw_rotation, raw_tapering,
           translation):
    N = raw_scale.shape[0]
    M = points.shape[1]
    MB = 1024
    grid = (pl.cdiv(M, MB),)

    pk = pl.pallas_call(
        _prep_kernel,
        out_shape=jax.ShapeDtypeStruct((N, 21), jnp.float32),
    )(raw_scale, raw_exponents, raw_rotation, raw_tapering, translation)

    full = lambda shape: pl.BlockSpec(shape, lambda i: (0, 0))
    out = pl.pallas_call(
        _sdf_block_kernel,
        grid=grid,
        in_specs=[
            pl.BlockSpec((3, MB), lambda i: (0, i)),
            full((N, 21)),
        ],
        out_specs=pl.BlockSpec((N, MB), lambda i: (0, i)),
        out_shape=jax.ShapeDtypeStruct((N, M), jnp.float32),
        compiler_params=pltpu.CompilerParams(
            dimension_semantics=("parallel",),
        ),
    )(points, pk)
    return out


# speedup vs baseline: 1.0597x; 1.0020x over previous
import functools

import jax
import jax.numpy as jnp
from jax.experimental import pallas as pl
from jax.experimental.pallas import tpu as pltpu

_MINE, _MAXE = 0.1, 1.9
_TRUNC = 0.1
_EPS = 1e-6


def _prep_kernel(scale_ref, exps_ref, rot_ref, taper_ref, trans_ref,
                 out_ref):
    scale = jnp.exp(scale_ref[...]) + 1e-6
    inv_s = 1.0 / scale
    isx = inv_s[:, 0:1]
    isy = inv_s[:, 1:2]
    isz = inv_s[:, 2:3]

    e = jax.nn.sigmoid(exps_ref[...]) * (_MAXE - _MINE) + _MINE
    e1 = e[:, 0:1]
    e2 = e[:, 1:2]
    p2 = 2.0 / e2
    p21 = e2 / e1
    p1 = 2.0 / e1
    ph = -0.5 * e1

    q = rot_ref[...]
    q = q / (jnp.sqrt(jnp.sum(q * q, axis=-1, keepdims=True)) + 1e-12)
    qw = q[:, 0:1]
    qx = q[:, 1:2]
    qy = q[:, 2:3]
    qz = q[:, 3:4]
    r00 = 1 - 2 * (qy * qy + qz * qz)
    r01 = 2 * (qx * qy - qw * qz)
    r02 = 2 * (qx * qz + qw * qy)
    r10 = 2 * (qx * qy + qw * qz)
    r11 = 1 - 2 * (qx * qx + qz * qz)
    r12 = 2 * (qy * qz - qw * qx)
    r20 = 2 * (qx * qz - qw * qy)
    r21 = 2 * (qy * qz + qw * qx)
    r22 = 1 - 2 * (qx * qx + qy * qy)

    taper = jnp.tanh(taper_ref[...])
    cx = taper[:, 0:1] * isz
    cy = taper[:, 1:2] * isz

    tx = trans_ref[:, 0:1]
    ty = trans_ref[:, 1:2]
    tz = trans_ref[:, 2:3]

    out_ref[...] = jnp.concatenate([
        r00, r10, r20, r01, r11, r21, r02, r12, r22,
        tx, ty, tz, isx * isx, isy * isy, isz * isz,
        1.0 / e2, p21, 1.0 / e1, ph, cx, cy,
    ], axis=1)


_CH = 32


def _sdf_block_kernel(points_ref, pk_ref, out_ref):
    pk = pk_ref[...]
    px = points_ref[0:1, :]
    py = points_ref[1:2, :]
    pz = points_ref[2:3, :]

    n = pk.shape[0]
    for k in range(0, n, _CH):
        col = lambda j: pk[k:k + _CH, j:j + 1]
        r00, r10, r20 = col(0), col(1), col(2)
        r01, r11, r21 = col(3), col(4), col(5)
        r02, r12, r22 = col(6), col(7), col(8)
        tx, ty, tz = col(9), col(10), col(11)
        isx2, isy2, isz2 = col(12), col(13), col(14)
        q2, p21, q1, ph = col(15), col(16), col(17), col(18)
        cx, cy = col(19), col(20)

        d0 = px - tx
        d1 = py - ty
        d2 = pz - tz
        x0 = r00 * d0 + r10 * d1 + r20 * d2
        x1 = r01 * d0 + r11 * d1 + r21 * d2
        x2 = r02 * d0 + r12 * d1 + r22 * d2

        sq0 = x0 * x0
        sq1 = x1 * x1
        sq2 = x2 * x2
        r = jnp.sqrt(sq0 + sq1 + sq2)

        gx = cx * x2 + 1.0
        gy = cy * x2 + 1.0

        lx = jnp.log2(sq0 * isx2) - jnp.log2(gx * gx)
        ly = jnp.log2(sq1 * isy2) - jnp.log2(gy * gy)
        lz = jnp.log2(sq2 * isz2)

        A = jnp.exp2(q2 * lx) + jnp.exp2(q2 * ly)
        B = jnp.exp2(p21 * jnp.log2(A)) + jnp.exp2(q1 * lz)
        sdf = r * (1.0 - jnp.exp2(ph * jnp.log2(B)))
        out_ref[k:k + _CH, :] = jnp.clip(sdf, -_TRUNC, _TRUNC)


@functools.partial(jax.jit, static_argnames=())
def kernel(points, raw_scale, raw_exponents, raw_rotation, raw_tapering,
           translation):
    N = raw_scale.shape[0]
    M = points.shape[1]
    MB = 1024
    grid = (pl.cdiv(M, MB),)

    pk = pl.pallas_call(
        _prep_kernel,
        out_shape=jax.ShapeDtypeStruct((N, 21), jnp.float32),
    )(raw_scale, raw_exponents, raw_rotation, raw_tapering, translation)

    full = lambda shape: pl.BlockSpec(shape, lambda i: (0, 0))
    out = pl.pallas_call(
        _sdf_block_kernel,
        grid=grid,
        in_specs=[
            pl.BlockSpec((3, MB), lambda i: (0, i)),
            full((N, 21)),
        ],
        out_specs=pl.BlockSpec((N, MB), lambda i: (0, i)),
        out_shape=jax.ShapeDtypeStruct((N, M), jnp.float32),
        compiler_params=pltpu.CompilerParams(
            dimension_semantics=("parallel",),
        ),
    )(points, pk)
    return out
